# baseline (device time: 19594 ns/iter reference)
import jax
import jax.numpy as jnp
from jax import lax
from jax.experimental import pallas as pl
from jax.experimental.pallas import tpu as pltpu

K = 16


def kernel(x):
    _, m, n = x.shape
    half = n // 2
    qm = m // 4
    cm = qm // K

    n_x = K + K // 2
    ay0, az0, b0 = n_x, n_x + K, n_x + 2 * K
    n_sem = b0 + K // 2

    def body(x_ref, out_ref, p_ref, ssem, rsem):
        my_x = lax.axis_index("x")
        my_y = lax.axis_index("y")
        my_z = lax.axis_index("z")
        x_nbr = (1 - my_x, my_y, my_z)
        y_nbr = (my_x, 1 - my_y, my_z)
        z_nbr = (my_x, my_y, 1 - my_z)

        q = 2 * my_y + my_z
        qy = 2 * (1 - my_y) + my_z
        qz = 2 * my_y + (1 - my_z)
        qd = 2 * (1 - my_y) + (1 - my_z)

        mycols = pl.ds(my_x * half, half)

        barrier_sem = pltpu.get_barrier_semaphore()
        for nbr in (x_nbr, y_nbr, z_nbr):
            pl.semaphore_signal(
                barrier_sem, inc=1, device_id=nbr,
                device_id_type=pl.DeviceIdType.MESH,
            )
        pl.semaphore_wait(barrier_sem, 3)

        def rcopy(slot, src, dst, dev):
            return pltpu.make_async_remote_copy(
                src_ref=src, dst_ref=dst,
                send_sem=ssem.at[slot], recv_sem=rsem.at[slot],
                device_id=dev, device_id_type=pl.DeviceIdType.MESH,
            )

        pcols = pl.ds((1 - my_x) * half, half)
        xr = [
            rcopy(c, x_ref.at[0, pl.ds(q * qm + c * cm, cm), pcols],
                  p_ref.at[c], x_nbr)
            for c in range(K)
        ] + [
            rcopy(K + c, x_ref.at[0, pl.ds(qd * qm + c * cm, cm), pcols],
                  p_ref.at[K + c], x_nbr)
            for c in range(K // 2)
        ]
        for r in xr:
            r.start()

        a_rdmas = []
        for c in range(K):
            xr[c].wait_recv()
            rows = pl.ds(q * qm + c * cm, cm)
            out_ref[rows, :] = x_ref[0, rows, mycols] + p_ref[c, :, :]
            ay = rcopy(ay0 + c, out_ref.at[rows, :], out_ref.at[rows, :], y_nbr)
            az = rcopy(az0 + c, out_ref.at[rows, :], out_ref.at[rows, :], z_nbr)
            ay.start()
            az.start()
            a_rdmas.append((ay, az))

        b_rdmas = []
        for i, c in enumerate(range(K // 2, 3 * K // 4)):
            a_rdmas[c][1].wait_recv()
            rows = pl.ds(qz * qm + c * cm, cm)
            by = rcopy(b0 + i, out_ref.at[rows, :], out_ref.at[rows, :], y_nbr)
            by.start()
            b_rdmas.append(by)
        for i, c in enumerate(range(3 * K // 4, K)):
            a_rdmas[c][0].wait_recv()
            rows = pl.ds(qy * qm + c * cm, cm)
            bz = rcopy(b0 + K // 4 + i, out_ref.at[rows, :],
                       out_ref.at[rows, :], z_nbr)
            bz.start()
            b_rdmas.append(bz)

        for c in range(K // 2):
            xr[K + c].wait_recv()
            rows = pl.ds(qd * qm + c * cm, cm)
            out_ref[rows, :] = x_ref[0, rows, mycols] + p_ref[K + c, :, :]

        for c in range(K):
            if not (3 * K // 4 <= c < K):
                a_rdmas[c][0].wait_recv()
            if not (K // 2 <= c < 3 * K // 4):
                a_rdmas[c][1].wait_recv()
        for b in b_rdmas:
            b.wait_recv()
        for r in xr:
            r.wait_send()
        for ay, az in a_rdmas:
            ay.wait_send()
            az.wait_send()
        for b in b_rdmas:
            b.wait_send()

    return pl.pallas_call(
        body,
        out_shape=jax.ShapeDtypeStruct((m, half), jnp.float32),
        in_specs=[pl.BlockSpec(memory_space=pltpu.VMEM)],
        out_specs=pl.BlockSpec(memory_space=pltpu.VMEM),
        scratch_shapes=[
            pltpu.VMEM((n_x, cm, half), jnp.float32),
            pltpu.SemaphoreType.DMA((n_sem,)),
            pltpu.SemaphoreType.DMA((n_sem,)),
        ],
        compiler_params=pltpu.CompilerParams(collective_id=0),
    )(x)
